# baseline (speedup 1.0000x reference)
import functools

import jax
import jax.numpy as jnp
from jax import lax
from jax.experimental import pallas as pl
from jax.experimental.pallas import tpu as pltpu
from jax.experimental.pallas import tpu_sc as plsc

N = 10000
E = 320000
D = 128
H = 128
C = 4096

_NC = 2
_NS = 16
_NW = _NC * _NS

_NPAD = 10240
_RPW = _NPAD // _NW
_DUMMY = _RPW
_SUB = 448
_CAP = 32 * _SUB
_GCH = _CAP // 128
_SCCH = 2000
_EPW = E // _NW
_NSC = _EPW // _SCCH


def _scalar(x):
    if getattr(x, "ndim", 0):
        return lax.reduce_max(x, axes=(0,))
    return x


@functools.lru_cache(maxsize=None)
def _build_sc_bucket():
    mesh = plsc.VectorSubcoreMesh(core_axis_name="c", subcore_axis_name="s")

    @functools.partial(
        pl.kernel,
        out_type=(jax.ShapeDtypeStruct((_NW, _CAP), jnp.int32),
                  jax.ShapeDtypeStruct((_NW, _CAP), jnp.int32)),
        mesh=mesh,
        compiler_params=pltpu.CompilerParams(needs_layout_passes=False),
        scratch_types=[
            pltpu.VMEM((_SCCH,), jnp.int32),
            pltpu.VMEM((_SCCH,), jnp.int32),
            pltpu.VMEM((_CAP + 16,), jnp.int32),
            pltpu.VMEM((_CAP + 16,), jnp.int32),
            pltpu.VMEM((48,), jnp.int32),
        ],
    )
    def bucket(src_hbm, dst_hbm, slist_hbm, dlist_hbm,
               dbuf_v, sbuf_v, sout_v, dout_v, cnt_v):
        c = lax.axis_index("c")
        s = lax.axis_index("s")
        wid = s * _NC + c
        base_e = wid * _EPW

        iota16 = lax.iota(jnp.int32, 16)
        lane0 = iota16 == 0
        zero16 = jnp.zeros((16,), jnp.int32)
        dummy16 = jnp.full((16,), _DUMMY, jnp.int32)

        def prefill(v, carry):
            sout_v[pl.ds(v * 16, 16)] = zero16
            dout_v[pl.ds(v * 16, 16)] = dummy16
            return carry

        lax.fori_loop(0, (_CAP + 16) // 16, prefill, 0)
        for q in range(3):
            cnt_v[pl.ds(q * 16, 16)] = zero16

        def scan_chunk(k, carry):
            off = base_e + k * _SCCH
            pltpu.sync_copy(dst_hbm.at[pl.ds(off, _SCCH)], dbuf_v)
            pltpu.sync_copy(src_hbm.at[pl.ds(off, _SCCH)], sbuf_v)

            def edge(i, carry):
                ii = jnp.full((16,), i, jnp.int32)
                dsp = plsc.load_gather(dbuf_v, [ii])
                ssp = plsc.load_gather(sbuf_v, [ii])
                b = lax.shift_right_logical(dsp * 52429, 24)
                dl = dsp - b * _RPW
                cnt = jnp.minimum(plsc.load_gather(cnt_v, [b]), _SUB - 1)
                pos = b * _SUB + cnt
                tpos = jnp.where(lane0, pos, _CAP + iota16)
                plsc.store_scatter(sout_v, [tpos], ssp)
                plsc.store_scatter(dout_v, [tpos], dl)
                cpos = jnp.where(lane0, b, 32 + iota16)
                plsc.store_scatter(cnt_v, [cpos], cnt + 1)
                return carry

            lax.fori_loop(0, _SCCH, edge, 0)
            return carry

        lax.fori_loop(0, _NSC, scan_chunk, 0)
        pltpu.sync_copy(sout_v.at[pl.ds(0, _CAP)], slist_hbm.at[wid])
        pltpu.sync_copy(dout_v.at[pl.ds(0, _CAP)], dlist_hbm.at[wid])

    return bucket


@functools.lru_cache(maxsize=None)
def _build_sc_segment_sum():
    mesh = plsc.VectorSubcoreMesh(core_axis_name="c", subcore_axis_name="s")

    @functools.partial(
        pl.kernel,
        out_type=jax.ShapeDtypeStruct((_NPAD, H), jnp.float32),
        mesh=mesh,
        compiler_params=pltpu.CompilerParams(needs_layout_passes=False),
        scratch_types=[
            pltpu.VMEM((_CAP,), jnp.int32),
            pltpu.VMEM((_CAP,), jnp.int32),
            pltpu.VMEM((128, H), jnp.float32),
            pltpu.VMEM((128, H), jnp.float32),
            pltpu.VMEM((_RPW + 8, H), jnp.float32),
            pltpu.SemaphoreType.DMA,
            pltpu.SemaphoreType.DMA,
        ],
    )
    def seg(h_hbm, slist_hbm, dlist_hbm, out_hbm,
            slist_v, dlist_v, bufa_v, bufb_v, acc_v, sema, semb):
        c = lax.axis_index("c")
        s = lax.axis_index("s")
        wid = s * _NC + c

        zrow = jnp.zeros((16,), jnp.float32)

        def zbody(r, carry):
            for f in range(8):
                acc_v[r, pl.ds(f * 16, 16)] = zrow
            return carry

        lax.fori_loop(0, _RPW + 8, zbody, 0)

        pltpu.sync_copy(slist_hbm.at[wid], slist_v)
        pltpu.sync_copy(dlist_hbm.at[wid], dlist_v)

        def gstart(kk, buf, sem):
            pltpu.async_copy(h_hbm.at[slist_v.at[pl.ds(kk * 128, 128)]],
                             buf, sem)

        def gwait(kk, buf, sem):
            pltpu.make_async_copy(
                h_hbm.at[slist_v.at[pl.ds(kk * 128, 128)]], buf, sem).wait()

        iota16 = lax.iota(jnp.int32, 16)
        cols = [f * 16 + iota16 for f in range(8)]

        def accumulate(kk, buf):
            def edge(i, carry):
                rl = plsc.load_gather(
                    dlist_v, [jnp.full((16,), kk * 128 + i, jnp.int32)])
                for f in range(8):
                    plsc.addupdate_scatter(acc_v, [rl, cols[f]],
                                           buf[i, pl.ds(f * 16, 16)])
                return carry

            lax.fori_loop(0, 128, edge, 0)

        gstart(0, bufa_v, sema)

        def body(t, carry):
            k0 = 2 * t
            k1 = 2 * t + 1
            gstart(k1, bufb_v, semb)
            gwait(k0, bufa_v, sema)
            accumulate(k0, bufa_v)

            @pl.when(k1 + 1 < _GCH)
            def _():
                gstart(k1 + 1, bufa_v, sema)

            gwait(k1, bufb_v, semb)
            accumulate(k1, bufb_v)
            return carry

        lax.fori_loop(0, _GCH // 2, body, 0)

        pltpu.sync_copy(acc_v.at[pl.ds(0, _RPW)],
                        out_hbm.at[pl.ds(wid * _RPW, _RPW)])

    return seg


def _sc_bucket(src, dst):
    return _build_sc_bucket()(src, dst)


def _sc_segment_sum(h, slist, dlist):
    return _build_sc_segment_sum()(h, slist, dlist)


_GIDX = 384
_GTOT = _GIDX * _NW


@functools.lru_cache(maxsize=None)
def _build_sc_gather():
    mesh = plsc.VectorSubcoreMesh(core_axis_name="c", subcore_axis_name="s")

    @functools.partial(
        pl.kernel,
        out_type=jax.ShapeDtypeStruct((_GTOT, H), jnp.float32),
        mesh=mesh,
        scratch_types=[
            pltpu.VMEM((3, 128), jnp.int32),
            pltpu.VMEM((128, H), jnp.float32),
            pltpu.SemaphoreType.DMA,
        ],
    )
    def gat(h_hbm, idx_hbm, out_hbm, idx_v, rows_v, sem):
        c = lax.axis_index("c")
        s = lax.axis_index("s")
        wid = s * _NC + c
        pltpu.sync_copy(idx_hbm.at[wid], idx_v)
        for j in range(3):
            pltpu.async_copy(h_hbm.at[idx_v.at[j]], rows_v, sem).wait()
            pltpu.sync_copy(rows_v,
                            out_hbm.at[pl.ds(wid * _GIDX + j * 128, 128)])

    return gat


def _sc_gather(h, idx):
    return _build_sc_gather()(h, idx)


def _gin_mlp_body(h_ref, a_ref, w1_ref, b1_ref, w2_ref, b2_ref, o_ref):
    z = h_ref[...] + a_ref[...]
    m = jnp.dot(z, w1_ref[...], preferred_element_type=jnp.float32)
    m = jnp.maximum(m + b1_ref[...], 0.0)
    o_ref[...] = (jnp.dot(m, w2_ref[...], preferred_element_type=jnp.float32)
                  + b2_ref[...])


_MLP_BLK = 1000


def _tc_gin_mlp(h, a, w1, b1, w2, b2):
    grid = (N // _MLP_BLK,)
    row_spec = pl.BlockSpec((_MLP_BLK, H), lambda i: (i, 0))
    full = pl.BlockSpec((H, H), lambda i: (0, 0))
    vec = pl.BlockSpec((1, H), lambda i: (0, 0))
    return pl.pallas_call(
        _gin_mlp_body,
        grid=grid,
        in_specs=[row_spec, row_spec, full, vec, full, vec],
        out_specs=row_spec,
        out_shape=jax.ShapeDtypeStruct((N, H), jnp.float32),
    )(h, a, w1, b1.reshape(1, H), w2, b2.reshape(1, H))


def _score_body(t_ref, xu_ref, xv_ref, fu_ref, fv_ref, temb_ref,
                w1a_ref, w1b_ref, w1c_ref, w1d_ref, w1e_ref, b1_ref,
                w2_ref, b2_ref, o_ref):
    tv = t_ref[0]
    temb = temb_ref[pl.ds(tv, 1), :]
    fu = fu_ref[...]
    fv = fv_ref[...]
    dot = lambda a, b: jnp.dot(a, b, preferred_element_type=jnp.float32)
    base = (dot(fu + fv, w1a_ref[...])
            + dot(jnp.abs(fu - fv), w1b_ref[...])
            + dot(temb, w1e_ref[...])
            + b1_ref[...])
    xu = xu_ref[...]
    xv = xv_ref[...]
    m = dot(xu + xv, w1c_ref[...]) + dot(jnp.abs(xu - xv), w1d_ref[...])
    m = jnp.maximum(m + base, 0.0)
    o_ref[...] = dot(m, w2_ref[...]) + b2_ref[...]


def _tc_score(t, xu, xv, fu, fv, t_embed_w, edge_W1, edge_b1, edge_W2,
              edge_b2):
    tarr = jnp.asarray(t, jnp.int32).reshape(1)
    w1a = edge_W1[0:H]
    w1b = edge_W1[H:2 * H]
    w1c = edge_W1[2 * H:3 * H]
    w1d = edge_W1[3 * H:4 * H]
    w1e = edge_W1[4 * H:5 * H]
    out = pl.pallas_call(
        _score_body,
        in_specs=[
            pl.BlockSpec(memory_space=pltpu.SMEM),
            pl.BlockSpec((C, H), lambda: (0, 0)),
            pl.BlockSpec((C, H), lambda: (0, 0)),
            pl.BlockSpec((1, H), lambda: (0, 0)),
            pl.BlockSpec((1, H), lambda: (0, 0)),
            pl.BlockSpec(((1000 + 1), H), lambda: (0, 0)),
            pl.BlockSpec((H, H), lambda: (0, 0)),
            pl.BlockSpec((H, H), lambda: (0, 0)),
            pl.BlockSpec((H, H), lambda: (0, 0)),
            pl.BlockSpec((H, H), lambda: (0, 0)),
            pl.BlockSpec((H, H), lambda: (0, 0)),
            pl.BlockSpec((1, H), lambda: (0, 0)),
            pl.BlockSpec((H, 1), lambda: (0, 0)),
            pl.BlockSpec((1, 1), lambda: (0, 0)),
        ],
        out_specs=pl.BlockSpec((C, 1), lambda: (0, 0)),
        out_shape=jax.ShapeDtypeStruct((C, 1), jnp.float32),
    )(tarr, xu, xv, fu, fv, t_embed_w, w1a, w1b, w1c, w1d, w1e,
      edge_b1.reshape(1, H), edge_W2, edge_b2.reshape(1, 1))
    return out.reshape(-1)


def kernel(x, edge_index, first_edge, candidate_edges, t,
           gin_W1_0, gin_b1_0, gin_W2_0, gin_b2_0,
           gin_W1_1, gin_b1_1, gin_W2_1, gin_b2_1,
           gin_W1_2, gin_b1_2, gin_W2_2, gin_b2_2,
           t_embed_w, edge_W1, edge_b1, edge_W2, edge_b2):
    ssub, dsub = _sc_bucket(edge_index[0], edge_index[1])
    slist = ssub.reshape(_NW, _NW, _SUB).transpose(1, 0, 2).reshape(_NW, _CAP)
    dlist = dsub.reshape(_NW, _NW, _SUB).transpose(1, 0, 2).reshape(_NW, _CAP)

    layers = [(gin_W1_0, gin_b1_0, gin_W2_0, gin_b2_0),
              (gin_W1_1, gin_b1_1, gin_W2_1, gin_b2_1),
              (gin_W1_2, gin_b1_2, gin_W2_2, gin_b2_2)]
    h = x
    for w1, b1, w2, b2 in layers:
        agg = _sc_segment_sum(h, slist, dlist)
        h = _tc_gin_mlp(h, agg[:N], w1, b1, w2, b2)

    idx_all = jnp.concatenate([
        candidate_edges[:, 0], candidate_edges[:, 1], first_edge,
        jnp.zeros((_GTOT - 2 * C - 2,), jnp.int32),
    ]).reshape(_NW, 3, 128)
    g = _sc_gather(h, idx_all)
    xu = g[0:C]
    xv = g[C:2 * C]
    fu = g[2 * C:2 * C + 1]
    fv = g[2 * C + 1:2 * C + 2]
    return _tc_score(t, xu, xv, fu, fv, t_embed_w, edge_W1, edge_b1,
                     edge_W2, edge_b2)
